# baseline (speedup 1.0000x reference)
import functools

import jax
import jax.numpy as jnp
from jax import lax
from jax.experimental import pallas as pl
from jax.experimental.pallas import tpu as pltpu
from jax.experimental.pallas import tpu_sc as plsc

B, G, D = 16, 20000, 128
S, L = 8, 64
SET_STRIDE = 100
LANES = 16
NCH = D // LANES
NUM_CORES = 2
NUM_SUBCORES = 16
NW = NUM_CORES * NUM_SUBCORES
BP = B // (NW // S)


def _zeros():
    return tuple(jnp.zeros((LANES,), jnp.float32) for _ in range(NCH))


def _agg_body(gene_hbm, attn_hbm, out_hbm, attn_v, e_v, gene_v, out_v,
              sem_a, sem_g0, sem_g1, sem_g2, sem_g3):
    cid = lax.axis_index("c")
    sid = lax.axis_index("s")
    wid = sid * NUM_CORES + cid
    set_id = wid % S
    b_base = (wid // S) * BP

    gene_sems = (sem_g0, sem_g1, sem_g2, sem_g3)
    cp_a = pltpu.async_copy(attn_hbm.at[set_id], attn_v, sem_a)
    cps = [
        pltpu.async_copy(
            gene_hbm.at[b_base + i, pl.ds(set_id * SET_STRIDE, L)],
            gene_v.at[i], gene_sems[i])
        for i in range(BP)
    ]

    cp_a.wait()

    def exp_body(l, denoms):
        new = []
        for c in range(NCH):
            w = attn_v[l, pl.ds(c * LANES, LANES)]
            e = jnp.exp(w)
            e_v[l, pl.ds(c * LANES, LANES)] = e
            new.append(denoms[c] + e)
        return tuple(new)

    denoms = lax.fori_loop(0, L, exp_body, _zeros())
    recips = tuple(1.0 / d for d in denoms)

    for i in range(BP):
        cps[i].wait()

        def acc_body(l, accs, i=i):
            return tuple(
                accs[c] + e_v[l, pl.ds(c * LANES, LANES)]
                * gene_v[i, l, pl.ds(c * LANES, LANES)]
                for c in range(NCH))

        accs = lax.fori_loop(0, L, acc_body, _zeros())
        for c in range(NCH):
            out_v[i, pl.ds(c * LANES, LANES)] = accs[c] * recips[c]
        pltpu.sync_copy(out_v.at[i], out_hbm.at[b_base + i, set_id])


@functools.lru_cache(maxsize=None)
def _build_agg():
    return pl.kernel(
        _agg_body,
        out_type=jax.ShapeDtypeStruct((B, S, D), jnp.float32),
        mesh=plsc.VectorSubcoreMesh(core_axis_name="c", subcore_axis_name="s",
                                    num_cores=NUM_CORES,
                                    num_subcores=NUM_SUBCORES),
        scratch_types=[
            pltpu.VMEM((L, D), jnp.float32),
            pltpu.VMEM((L, D), jnp.float32),
            pltpu.VMEM((BP, L, D), jnp.float32),
            pltpu.VMEM((BP, D), jnp.float32),
            pltpu.SemaphoreType.DMA,
            pltpu.SemaphoreType.DMA,
            pltpu.SemaphoreType.DMA,
            pltpu.SemaphoreType.DMA,
            pltpu.SemaphoreType.DMA,
        ],
        compiler_params=pltpu.CompilerParams(use_tc_tiling_on_sc=False),
    )


def kernel(gene_features, attn_weights):
    return _build_agg()(gene_features, attn_weights)


# speedup vs baseline: 1.0793x; 1.0793x over previous
import functools

import jax
import jax.numpy as jnp
from jax import lax
from jax.experimental import pallas as pl
from jax.experimental.pallas import tpu as pltpu
from jax.experimental.pallas import tpu_sc as plsc

B, G, D = 16, 20000, 128
S, L = 8, 64
SET_STRIDE = 100
LANES = 16
NCH = D // LANES
NUM_CORES = 2
NUM_SUBCORES = 16
NW = NUM_CORES * NUM_SUBCORES
BP = B // (NW // S)


def _agg_body(gene_hbm, attn_hbm, out_hbm, attn_v, gene_v, out_v,
              sem_a, sem_g, sem_o):
    cid = lax.axis_index("c")
    sid = lax.axis_index("s")
    wid = sid * NUM_CORES + cid
    set_id = wid % S
    b_base = (wid // S) * BP

    cp_a = pltpu.async_copy(attn_hbm.at[set_id], attn_v, sem_a)
    cp_g = pltpu.async_copy(
        gene_hbm.at[pl.ds(b_base, BP), pl.ds(set_id * SET_STRIDE, L)],
        gene_v, sem_g)
    cp_a.wait()
    cp_g.wait()

    def body(l, carry):
        denoms, accs = carry
        new_d = []
        new_a = []
        for c in range(NCH):
            w = attn_v[l, pl.ds(c * LANES, LANES)]
            e = jnp.exp(w)
            new_d.append(denoms[c] + e)
            for i in range(BP):
                g = gene_v[i, l, pl.ds(c * LANES, LANES)]
                new_a.append(accs[c * BP + i] + e * g)
        return tuple(new_d), tuple(new_a)

    zeros = tuple(jnp.zeros((LANES,), jnp.float32) for _ in range(NCH))
    azeros = tuple(jnp.zeros((LANES,), jnp.float32) for _ in range(NCH * BP))
    denoms, accs = lax.fori_loop(0, L, body, (zeros, azeros))

    for c in range(NCH):
        r = 1.0 / denoms[c]
        for i in range(BP):
            out_v[i, pl.ds(c * LANES, LANES)] = accs[c * BP + i] * r

    pltpu.async_copy(out_v, out_hbm.at[pl.ds(b_base, BP), set_id],
                     sem_o).wait()


@functools.lru_cache(maxsize=None)
def _build_agg():
    return pl.kernel(
        _agg_body,
        out_type=jax.ShapeDtypeStruct((B, S, D), jnp.float32),
        mesh=plsc.VectorSubcoreMesh(core_axis_name="c", subcore_axis_name="s",
                                    num_cores=NUM_CORES,
                                    num_subcores=NUM_SUBCORES),
        scratch_types=[
            pltpu.VMEM((L, D), jnp.float32),
            pltpu.VMEM((BP, L, D), jnp.float32),
            pltpu.VMEM((BP, D), jnp.float32),
            pltpu.SemaphoreType.DMA,
            pltpu.SemaphoreType.DMA,
            pltpu.SemaphoreType.DMA,
        ],
        compiler_params=pltpu.CompilerParams(use_tc_tiling_on_sc=False,
                                             skip_device_barrier=True),
    )


def kernel(gene_features, attn_weights):
    return _build_agg()(gene_features, attn_weights)
